# baseline (speedup 1.0000x reference)
import jax
import jax.numpy as jnp
from jax.experimental import pallas as pl
from jax.experimental.pallas import tpu as pltpu

N = 10000
DEG = 32
D_MSG = 128
D_FEAT = 128
D_HID = 128
IN = D_MSG + D_FEAT + D_HID
H = 256
OUT = 128

BN = 1000


def _fused_body(mb_ref, nf_ref, nh_ref, w1_ref, b1_ref,
                w2_ref, b2_ref, w3_ref, b3_ref, o_ref):
    msum = jnp.sum(mb_ref[...], axis=1)
    h = (jnp.dot(msum, w1_ref[0:D_MSG, :],
                 preferred_element_type=jnp.float32)
         + jnp.dot(nf_ref[...], w1_ref[D_MSG:D_MSG + D_FEAT, :],
                   preferred_element_type=jnp.float32)
         + jnp.dot(nh_ref[...], w1_ref[D_MSG + D_FEAT:IN, :],
                   preferred_element_type=jnp.float32)
         + b1_ref[...][None, :])
    h = jnp.maximum(h, 0.0)
    h = jnp.dot(h, w2_ref[...], preferred_element_type=jnp.float32) + b2_ref[...][None, :]
    h = jnp.maximum(h, 0.0)
    o_ref[...] = jnp.dot(h, w3_ref[...], preferred_element_type=jnp.float32) + b3_ref[...][None, :]


@jax.jit
def kernel(mailbox, node_features, node_hidden_rep, W1, b1, W2, b2, W3, b3):
    grid = (N // BN,)

    return pl.pallas_call(
        _fused_body,
        grid=grid,
        in_specs=[
            pl.BlockSpec((BN, DEG, D_MSG), lambda i: (i, 0, 0)),
            pl.BlockSpec((BN, D_FEAT), lambda i: (i, 0)),
            pl.BlockSpec((BN, D_HID), lambda i: (i, 0)),
            pl.BlockSpec(W1.shape, lambda i: (0, 0)),
            pl.BlockSpec(b1.shape, lambda i: (0,)),
            pl.BlockSpec(W2.shape, lambda i: (0, 0)),
            pl.BlockSpec(b2.shape, lambda i: (0,)),
            pl.BlockSpec(W3.shape, lambda i: (0, 0)),
            pl.BlockSpec(b3.shape, lambda i: (0,)),
        ],
        out_specs=pl.BlockSpec((BN, OUT), lambda i: (i, 0)),
        out_shape=jax.ShapeDtypeStruct((N, OUT), jnp.float32),
        compiler_params=pltpu.CompilerParams(
            dimension_semantics=("parallel",),
        ),
    )(mailbox, node_features, node_hidden_rep, W1, b1, W2, b2, W3, b3)


# speedup vs baseline: 1.0083x; 1.0083x over previous
import jax
import jax.numpy as jnp
from jax.experimental import pallas as pl
from jax.experimental.pallas import tpu as pltpu

N = 10000
DEG = 32
D_MSG = 128
D_FEAT = 128
D_HID = 128
IN = D_MSG + D_FEAT + D_HID
H = 256
OUT = 128

BN = 400


def _fused_body(mb_ref, nf_ref, nh_ref, w1_ref, b1_ref,
                w2_ref, b2_ref, w3_ref, b3_ref, o_ref):
    msum = jnp.sum(mb_ref[...], axis=1)
    h = (jnp.dot(msum, w1_ref[0:D_MSG, :],
                 preferred_element_type=jnp.float32)
         + jnp.dot(nf_ref[...], w1_ref[D_MSG:D_MSG + D_FEAT, :],
                   preferred_element_type=jnp.float32)
         + jnp.dot(nh_ref[...], w1_ref[D_MSG + D_FEAT:IN, :],
                   preferred_element_type=jnp.float32)
         + b1_ref[...][None, :])
    h = jnp.maximum(h, 0.0)
    h = jnp.dot(h, w2_ref[...], preferred_element_type=jnp.float32) + b2_ref[...][None, :]
    h = jnp.maximum(h, 0.0)
    o_ref[...] = jnp.dot(h, w3_ref[...], preferred_element_type=jnp.float32) + b3_ref[...][None, :]


@jax.jit
def kernel(mailbox, node_features, node_hidden_rep, W1, b1, W2, b2, W3, b3):
    grid = (N // BN,)

    return pl.pallas_call(
        _fused_body,
        grid=grid,
        in_specs=[
            pl.BlockSpec((BN, DEG, D_MSG), lambda i: (i, 0, 0)),
            pl.BlockSpec((BN, D_FEAT), lambda i: (i, 0)),
            pl.BlockSpec((BN, D_HID), lambda i: (i, 0)),
            pl.BlockSpec(W1.shape, lambda i: (0, 0)),
            pl.BlockSpec(b1.shape, lambda i: (0,)),
            pl.BlockSpec(W2.shape, lambda i: (0, 0)),
            pl.BlockSpec(b2.shape, lambda i: (0,)),
            pl.BlockSpec(W3.shape, lambda i: (0, 0)),
            pl.BlockSpec(b3.shape, lambda i: (0,)),
        ],
        out_specs=pl.BlockSpec((BN, OUT), lambda i: (i, 0)),
        out_shape=jax.ShapeDtypeStruct((N, OUT), jnp.float32),
        compiler_params=pltpu.CompilerParams(
            dimension_semantics=("parallel",),
        ),
    )(mailbox, node_features, node_hidden_rep, W1, b1, W2, b2, W3, b3)
